# baseline (speedup 1.0000x reference)
import functools

import jax
import jax.numpy as jnp
from jax import lax
from jax.experimental import pallas as pl
from jax.experimental.pallas import tpu as pltpu
from jax.experimental.pallas import tpu_sc as plsc

NC = 2
NS = 16
L = 16
CHUNK = 128
G = 4
INIT_ROWS = 125


def _sc_kernel(e_per_tile, half, n, x, src, dst, vals, bias_h, out, acc, xsh,
               src_v, dst_v, vals_v, rows_v, binit_v, bias_v,
               sem_idx, sem_g, sem_s):
    cid = lax.axis_index("c")
    sid = lax.axis_index("s")
    n_chunks_tile = e_per_tile // CHUNK
    n_pairs = n_chunks_tile // (2 * G)
    chunk0 = sid * n_chunks_tile
    rpt = n // NS

    def idx_start(pset, b, chunk_id):
        off = pl.ds(chunk_id * CHUNK, CHUNK)
        pltpu.async_copy(src.at[off], src_v.at[pset, b], sem_idx.at[pset, b])
        pltpu.async_copy(dst.at[off], dst_v.at[pset, b], sem_idx.at[pset, b])
        pltpu.async_copy(vals.at[off], vals_v.at[pset, b],
                         sem_idx.at[pset, b])

    def idx_wait(pset, b):
        off = pl.ds(0, CHUNK)
        pltpu.make_async_copy(src.at[off], src_v.at[pset, b],
                              sem_idx.at[pset, b]).wait()
        pltpu.make_async_copy(dst.at[off], dst_v.at[pset, b],
                              sem_idx.at[pset, b]).wait()
        pltpu.make_async_copy(vals.at[off], vals_v.at[pset, b],
                              sem_idx.at[pset, b]).wait()

    def gather_start(pset, b):
        pltpu.async_copy(xsh.at[src_v.at[pset, b]], rows_v.at[b],
                         sem_g.at[b])

    def gather_wait(b):
        pltpu.make_async_copy(x.at[pl.ds(0, CHUNK), pl.ds(0, half)],
                              rows_v.at[b], sem_g.at[b]).wait()

    def scatter_start(pset, b):
        pltpu.async_copy(rows_v.at[b], acc.at[dst_v.at[pset, b]],
                         sem_s.at[b], add=True)

    def scatter_wait(b):
        pltpu.make_async_copy(rows_v.at[b], acc.at[pl.ds(0, CHUNK)],
                              sem_s.at[b]).wait()

    def scale(pset, b):
        def _grp(g, _):
            vv = vals_v[pset, b, pl.ds(g * L, L)]
            for k in range(L):
                i = g * L + k
                vk = vv[k]
                for f in range(half // L):
                    sl = pl.ds(f * L, L)
                    rows_v[b, i, sl] = rows_v[b, i, sl] * vk
            return 0

        lax.fori_loop(0, CHUNK // L, _grp, 0, unroll=False)

    for b in range(G):
        idx_start(0, b, chunk0 + b)

    rows = pl.ds(sid * rpt, rpt)
    for c in range(NC):
        @pl.when(cid == c)
        def _(c=c):
            pltpu.async_copy(x.at[rows, pl.ds(c * half, half)], xsh.at[rows],
                             sem_g.at[0])

    pltpu.sync_copy(bias_h, bias_v)

    def _binit_row(i, _):
        for f in range(half // L):
            binit_v[i, pl.ds(f * L, L)] = bias_v[pl.ds(cid * half + f * L, L)]
        return 0

    lax.fori_loop(0, INIT_ROWS, _binit_row, 0, unroll=False)
    pltpu.make_async_copy(x.at[rows, pl.ds(0, half)], xsh.at[rows],
                          sem_g.at[0]).wait()

    for k in range(rpt // INIT_ROWS):
        pltpu.sync_copy(
            binit_v, acc.at[pl.ds(sid * rpt + k * INIT_ROWS, INIT_ROWS)])
    plsc.subcore_barrier()

    def _pair(j, _):
        for phase in range(2):
            myset, nxtset = phase, 1 - phase
            g = 2 * j + phase
            for b in range(G):
                if phase == 0:
                    @pl.when(j > 0)
                    def _(b=b):
                        scatter_wait(b)
                else:
                    scatter_wait(b)
                idx_wait(myset, b)
                gather_start(myset, b)
            if phase == 0:
                for b in range(G):
                    idx_start(nxtset, b, chunk0 + (g + 1) * G + b)
            else:
                @pl.when(j + 1 < n_pairs)
                def _():
                    for b in range(G):
                        idx_start(nxtset, b, chunk0 + (g + 1) * G + b)
            for b in range(G):
                gather_wait(b)
                scale(myset, b)
                scatter_start(myset, b)
        return 0

    lax.fori_loop(0, n_pairs, _pair, 0, unroll=False)
    for b in range(G):
        scatter_wait(b)
    plsc.subcore_barrier()

    for c in range(NC):
        @pl.when(cid == c)
        def _():
            pltpu.sync_copy(acc.at[rows], out.at[rows, pl.ds(c * half, half)])


def kernel(input_feature, edge_index, edge_values, bias):
    n, d = input_feature.shape
    half = d // NC
    e = edge_index.shape[1]
    quantum = CHUNK * 2 * G
    e_per_tile = -(-e // (NS * quantum)) * quantum
    e_pad = e_per_tile * NS
    src = edge_index[0].astype(jnp.int32)
    dst = edge_index[1].astype(jnp.int32)
    vals = edge_values.astype(jnp.float32)
    if e_pad != e:
        src = jnp.pad(src, (0, e_pad - e))
        dst = jnp.pad(dst, (0, e_pad - e))
        vals = jnp.pad(vals, (0, e_pad - e))
    mesh = plsc.VectorSubcoreMesh(
        core_axis_name="c", subcore_axis_name="s", num_cores=NC,
        num_subcores=NS)
    return pl.kernel(
        functools.partial(_sc_kernel, e_per_tile, half, n),
        out_type=jax.ShapeDtypeStruct((n, d), jnp.float32),
        mesh=mesh,
        scratch_types=[
            pltpu.VMEM_SHARED((n, half), jnp.float32),
            pltpu.VMEM_SHARED((n, half), jnp.float32),
            pltpu.VMEM((2, G, CHUNK), jnp.int32),
            pltpu.VMEM((2, G, CHUNK), jnp.int32),
            pltpu.VMEM((2, G, CHUNK), jnp.float32),
            pltpu.VMEM((G, CHUNK, half), jnp.float32),
            pltpu.VMEM((INIT_ROWS, half), jnp.float32),
            pltpu.VMEM((d,), jnp.float32),
            pltpu.SemaphoreType.DMA((2, G)),
            pltpu.SemaphoreType.DMA((G,)),
            pltpu.SemaphoreType.DMA((G,)),
        ],
        compiler_params=pltpu.CompilerParams(use_tc_tiling_on_sc=False,
                                             needs_layout_passes=False),
    )(input_feature, src, dst, vals, bias)


# speedup vs baseline: 1.0277x; 1.0277x over previous
import functools

import jax
import jax.numpy as jnp
from jax import lax
from jax.experimental import pallas as pl
from jax.experimental.pallas import tpu as pltpu
from jax.experimental.pallas import tpu_sc as plsc

NC = 2
NS = 16
L = 16
CHUNK = 128
G = 4
INIT_ROWS = 125


def _sc_kernel(e_per_tile, half, n, x, src, dst, vals, bias_h, out, acc, xsh,
               src_v, dst_v, vals_v, rows_v, binit_v, bias_v,
               sem_idx, sem_g, sem_s):
    cid = lax.axis_index("c")
    sid = lax.axis_index("s")
    n_chunks_tile = e_per_tile // CHUNK
    n_pairs = n_chunks_tile // (2 * G)
    chunk0 = sid * n_chunks_tile
    rpt = n // NS

    def idx_start(pset, b, chunk_id):
        off = pl.ds(chunk_id * CHUNK, CHUNK)
        pltpu.async_copy(src.at[off], src_v.at[pset, b], sem_idx.at[pset, b])
        pltpu.async_copy(dst.at[off], dst_v.at[pset, b], sem_idx.at[pset, b])
        pltpu.async_copy(vals.at[off], vals_v.at[pset, b],
                         sem_idx.at[pset, b])

    def idx_wait(pset, b):
        off = pl.ds(0, CHUNK)
        pltpu.make_async_copy(src.at[off], src_v.at[pset, b],
                              sem_idx.at[pset, b]).wait()
        pltpu.make_async_copy(dst.at[off], dst_v.at[pset, b],
                              sem_idx.at[pset, b]).wait()
        pltpu.make_async_copy(vals.at[off], vals_v.at[pset, b],
                              sem_idx.at[pset, b]).wait()

    def gather_start(pset, b):
        pltpu.async_copy(xsh.at[src_v.at[pset, b]], rows_v.at[b],
                         sem_g.at[b])

    def gather_wait(b):
        pltpu.make_async_copy(x.at[pl.ds(0, CHUNK), pl.ds(0, half)],
                              rows_v.at[b], sem_g.at[b]).wait()

    def scatter_start(pset, b):
        pltpu.async_copy(rows_v.at[b], acc.at[dst_v.at[pset, b]],
                         sem_s.at[b], add=True)

    def scatter_wait(b):
        pltpu.make_async_copy(rows_v.at[b], acc.at[pl.ds(0, CHUNK)],
                              sem_s.at[b]).wait()

    def scale(pset, b):
        def _grp(g, _):
            vv = vals_v[pset, b, pl.ds(g * L, L)]
            for k in range(L):
                i = g * L + k
                vk = vv[k]
                for f in range(half // L):
                    sl = pl.ds(f * L, L)
                    rows_v[b, i, sl] = rows_v[b, i, sl] * vk
            return 0

        lax.fori_loop(0, CHUNK // L, _grp, 0, unroll=False)

    for b in range(G):
        idx_start(0, b, chunk0 + b)

    rows = pl.ds(sid * rpt, rpt)
    for c in range(NC):
        @pl.when(cid == c)
        def _():
            pltpu.sync_copy(x.at[rows, pl.ds(c * half, half)], xsh.at[rows])

    pltpu.sync_copy(bias_h, bias_v)

    def _binit_row(i, _):
        for f in range(half // L):
            binit_v[i, pl.ds(f * L, L)] = bias_v[pl.ds(cid * half + f * L, L)]
        return 0

    lax.fori_loop(0, INIT_ROWS, _binit_row, 0, unroll=False)

    for k in range(rpt // INIT_ROWS):
        pltpu.sync_copy(
            binit_v, acc.at[pl.ds(sid * rpt + k * INIT_ROWS, INIT_ROWS)])
    plsc.subcore_barrier()

    def _pair(j, _):
        for phase in range(2):
            myset, nxtset = phase, 1 - phase
            g = 2 * j + phase
            if phase == 0:
                @pl.when(j > 0)
                def _():
                    for b in range(G):
                        scatter_wait(b)
            else:
                for b in range(G):
                    scatter_wait(b)
            for b in range(G):
                idx_wait(myset, b)
                gather_start(myset, b)
            if phase == 0:
                for b in range(G):
                    idx_start(nxtset, b, chunk0 + (g + 1) * G + b)
            else:
                @pl.when(j + 1 < n_pairs)
                def _():
                    for b in range(G):
                        idx_start(nxtset, b, chunk0 + (g + 1) * G + b)
            for b in range(G):
                gather_wait(b)
                scale(myset, b)
                scatter_start(myset, b)
        return 0

    lax.fori_loop(0, n_pairs, _pair, 0, unroll=False)
    for b in range(G):
        scatter_wait(b)
    plsc.subcore_barrier()

    for c in range(NC):
        @pl.when(cid == c)
        def _():
            pltpu.sync_copy(acc.at[rows], out.at[rows, pl.ds(c * half, half)])


def kernel(input_feature, edge_index, edge_values, bias):
    n, d = input_feature.shape
    half = d // NC
    e = edge_index.shape[1]
    quantum = CHUNK * 2 * G
    e_per_tile = -(-e // (NS * quantum)) * quantum
    e_pad = e_per_tile * NS
    src = edge_index[0].astype(jnp.int32)
    dst = edge_index[1].astype(jnp.int32)
    vals = edge_values.astype(jnp.float32)
    if e_pad != e:
        src = jnp.pad(src, (0, e_pad - e))
        dst = jnp.pad(dst, (0, e_pad - e))
        vals = jnp.pad(vals, (0, e_pad - e))
    mesh = plsc.VectorSubcoreMesh(
        core_axis_name="c", subcore_axis_name="s", num_cores=NC,
        num_subcores=NS)
    return pl.kernel(
        functools.partial(_sc_kernel, e_per_tile, half, n),
        out_type=jax.ShapeDtypeStruct((n, d), jnp.float32),
        mesh=mesh,
        scratch_types=[
            pltpu.VMEM_SHARED((n, half), jnp.float32),
            pltpu.VMEM_SHARED((n, half), jnp.float32),
            pltpu.VMEM((2, G, CHUNK), jnp.int32),
            pltpu.VMEM((2, G, CHUNK), jnp.int32),
            pltpu.VMEM((2, G, CHUNK), jnp.float32),
            pltpu.VMEM((G, CHUNK, half), jnp.float32),
            pltpu.VMEM((INIT_ROWS, half), jnp.float32),
            pltpu.VMEM((d,), jnp.float32),
            pltpu.SemaphoreType.DMA((2, G)),
            pltpu.SemaphoreType.DMA((G,)),
            pltpu.SemaphoreType.DMA((G,)),
        ],
        compiler_params=pltpu.CompilerParams(use_tc_tiling_on_sc=False,
                                             needs_layout_passes=False),
    )(input_feature, src, dst, vals, bias)
